# baseline (speedup 1.0000x reference)
import functools

import jax
import jax.numpy as jnp
from jax import lax
from jax.experimental import pallas as pl
from jax.experimental.pallas import tpu as pltpu
from jax.experimental.pallas import tpu_sc as plsc

TOPK = 8
ROWS = 128
N = 32768
L = 16
NC = 2
NS = 16
NW = NC * NS
RPW = ROWS // NW
GROUPS = 32
GELEMS = N // GROUPS
GCHUNKS = GELEMS // L
NEG = float("-inf")
BIG = 2**30


def _neg():
    return jnp.full((L,), NEG, jnp.float32)


_MESH = plsc.VectorSubcoreMesh(core_axis_name="c", subcore_axis_name="s")


@functools.partial(
    pl.kernel,
    mesh=_MESH,
    compiler_params=pltpu.CompilerParams(needs_layout_passes=False),
    out_type=[
        jax.ShapeDtypeStruct((ROWS, L), jnp.float32),
        jax.ShapeDtypeStruct((ROWS, L), jnp.int32),
    ],
    scratch_types=[
        pltpu.VMEM((N,), jnp.float32),
        pltpu.VMEM((N,), jnp.float32),
        pltpu.VMEM((GROUPS, L), jnp.float32),
        pltpu.VMEM((RPW, L), jnp.float32),
        pltpu.VMEM((RPW, L), jnp.int32),
        pltpu.SemaphoreType.DMA,
        pltpu.SemaphoreType.DMA,
    ],
)
def _sc_topk(x_hbm, vals_hbm, idx_hbm, xv0, xv1, smax, vout, iout, sem0, sem1):
    wid = lax.axis_index("s") * NC + lax.axis_index("c")
    lane = lax.iota(jnp.int32, L)
    base_row = wid * RPW

    def row_compute(xv, r):
        def group_body(g, macc_ag):
            macc, agacc = macc_ag
            goff = g * GELEMS

            def ch_body(c, gms):
                g0, g1, g2, g3 = gms
                base = goff + c * (8 * L)
                g0 = jnp.maximum(g0, xv[pl.ds(base + 0 * L, L)])
                g1 = jnp.maximum(g1, xv[pl.ds(base + 1 * L, L)])
                g2 = jnp.maximum(g2, xv[pl.ds(base + 2 * L, L)])
                g3 = jnp.maximum(g3, xv[pl.ds(base + 3 * L, L)])
                g0 = jnp.maximum(g0, xv[pl.ds(base + 4 * L, L)])
                g1 = jnp.maximum(g1, xv[pl.ds(base + 5 * L, L)])
                g2 = jnp.maximum(g2, xv[pl.ds(base + 6 * L, L)])
                g3 = jnp.maximum(g3, xv[pl.ds(base + 7 * L, L)])
                return g0, g1, g2, g3

            g0, g1, g2, g3 = lax.fori_loop(
                0, GCHUNKS // 8, ch_body, (_neg(), _neg(), _neg(), _neg()))
            gm = jnp.maximum(jnp.maximum(g0, g1), jnp.maximum(g2, g3))
            smax[g, :] = gm
            upd = gm > macc
            macc = jnp.where(upd, gm, macc)
            agacc = jnp.where(upd, g, agacc)
            return macc, agacc

        ma, ag = lax.fori_loop(
            0, GROUPS, group_body,
            (_neg(), jnp.zeros((L,), jnp.int32)))

        vacc = jnp.zeros((L,), jnp.float32)
        iacc = jnp.zeros((L,), jnp.int32)
        for k in range(TOPK):
            mk = jnp.max(ma)

            gi = jnp.min(jnp.where(ma == mk, ag, BIG))
            gbase = gi * GELEMS

            def pos_body(c, pa):
                p0, p1, r0, r1, c0, c1 = pa
                base = gbase + c * (4 * L)
                pos = c * (4 * L) + lane
                v0 = xv[pl.ds(base + 0 * L, L)]
                v1 = xv[pl.ds(base + 1 * L, L)]
                v2 = xv[pl.ds(base + 2 * L, L)]
                v3 = xv[pl.ds(base + 3 * L, L)]
                e0 = v0 == mk
                e1 = v1 == mk
                e2 = v2 == mk
                e3 = v3 == mk
                p0 = jnp.minimum(p0, jnp.where(e0, pos + 0 * L, BIG))
                p1 = jnp.minimum(p1, jnp.where(e1, pos + 1 * L, BIG))
                p0 = jnp.minimum(p0, jnp.where(e2, pos + 2 * L, BIG))
                p1 = jnp.minimum(p1, jnp.where(e3, pos + 3 * L, BIG))
                r0 = jnp.maximum(r0, jnp.where(e0, NEG, v0))
                r1 = jnp.maximum(r1, jnp.where(e1, NEG, v1))
                r0 = jnp.maximum(r0, jnp.where(e2, NEG, v2))
                r1 = jnp.maximum(r1, jnp.where(e3, NEG, v3))
                c0 = c0 + e0.astype(jnp.int32)
                c1 = c1 + e1.astype(jnp.int32)
                c0 = c0 + e2.astype(jnp.int32)
                c1 = c1 + e3.astype(jnp.int32)
                return p0, p1, r0, r1, c0, c1

            bigv = jnp.full((L,), BIG, jnp.int32)
            zi = jnp.zeros((L,), jnp.int32)
            p0, p1, r0, r1, c0, c1 = lax.fori_loop(
                0, GCHUNKS // 4, pos_body,
                (bigv, bigv, _neg(), _neg(), zi, zi))
            e = jnp.min(jnp.minimum(p0, p1))

            vacc = jnp.where(lane == k, mk, vacc)
            iacc = jnp.where(lane == k, gbase + e, iacc)

            if k < TOPK - 1:
                el = e % L
                coff = gbase + e - el
                v = xv[pl.ds(coff, L)]
                xv[pl.ds(coff, L)] = jnp.where(lane == el, NEG, v)

                ct = (c0 + c1) - (lane == el).astype(jnp.int32)
                ru = jnp.maximum(r0, r1)
                smax[gi, :] = jnp.where(ct > 0, mk, ru)

                def remax_body(c, mbs):
                    b, bg = mbs
                    g = c * 4
                    for j in range(4):
                        gv = smax[g + j, :]
                        u = gv > b
                        b = jnp.where(u, gv, b)
                        bg = jnp.where(u, g + j, bg)
                    return b, bg

                ma, ag = lax.fori_loop(
                    0, GROUPS // 4, remax_body,
                    (_neg(), jnp.zeros((L,), jnp.int32)))

        vout[r, :] = vacc
        iout[r, :] = iacc

    pltpu.async_copy(x_hbm.at[base_row], xv0, sem0)
    for r in range(RPW):
        cur, sem_c = (xv0, sem0) if r % 2 == 0 else (xv1, sem1)
        nxt, sem_n = (xv1, sem1) if r % 2 == 0 else (xv0, sem0)
        pltpu.make_async_copy(x_hbm.at[base_row + r], cur, sem_c).wait()
        if r + 1 < RPW:
            pltpu.async_copy(x_hbm.at[base_row + r + 1], nxt, sem_n)
        row_compute(cur, r)

    pltpu.sync_copy(vout, vals_hbm.at[pl.ds(base_row, RPW)])
    pltpu.sync_copy(iout, idx_hbm.at[pl.ds(base_row, RPW)])


def _tc_norm_body(x_ref, m_ref, s_ref):
    xb = x_ref[...]
    m = jnp.max(xb, axis=1, keepdims=True)
    m_ref[...] = m
    s_ref[...] = jnp.sum(jnp.exp(xb - m), axis=1, keepdims=True)


_BR = 16


_tc_norm = pl.pallas_call(
    _tc_norm_body,
    grid=(ROWS // _BR,),
    in_specs=[pl.BlockSpec((_BR, N), lambda i: (i, 0))],
    out_specs=[
        pl.BlockSpec((_BR, 1), lambda i: (i, 0)),
        pl.BlockSpec((_BR, 1), lambda i: (i, 0)),
    ],
    out_shape=[
        jax.ShapeDtypeStruct((ROWS, 1), jnp.float32),
        jax.ShapeDtypeStruct((ROWS, 1), jnp.float32),
    ],
)


@jax.jit
def kernel(x):
    rawv, idx = _sc_topk(x)
    m, s = _tc_norm(x)
    vals = jnp.exp(rawv[:, :TOPK] - m) / s
    return vals, idx[:, :TOPK]


# speedup vs baseline: 1.0030x; 1.0030x over previous
import functools

import jax
import jax.numpy as jnp
from jax import lax
from jax.experimental import pallas as pl
from jax.experimental.pallas import tpu as pltpu
from jax.experimental.pallas import tpu_sc as plsc

TOPK = 8
ROWS = 128
N = 32768
L = 16
NC = 2
NS = 16
NW = NC * NS
RPW = ROWS // NW
GROUPS = 32
GELEMS = N // GROUPS
GCHUNKS = GELEMS // L
NEG = float("-inf")
BIG = 2**30


def _neg():
    return jnp.full((L,), NEG, jnp.float32)


_MESH = plsc.VectorSubcoreMesh(core_axis_name="c", subcore_axis_name="s")


@functools.partial(
    pl.kernel,
    mesh=_MESH,
    compiler_params=pltpu.CompilerParams(needs_layout_passes=False),
    out_type=[
        jax.ShapeDtypeStruct((ROWS, L), jnp.float32),
        jax.ShapeDtypeStruct((ROWS, L), jnp.int32),
    ],
    scratch_types=[
        pltpu.VMEM((N,), jnp.float32),
        pltpu.VMEM((N,), jnp.float32),
        pltpu.VMEM((GROUPS, L), jnp.float32),
        pltpu.VMEM((RPW, L), jnp.float32),
        pltpu.VMEM((RPW, L), jnp.int32),
        pltpu.SemaphoreType.DMA,
        pltpu.SemaphoreType.DMA,
    ],
)
def _sc_topk(x_hbm, vals_hbm, idx_hbm, xv0, xv1, smax, vout, iout, sem0, sem1):
    wid = lax.axis_index("s") * NC + lax.axis_index("c")
    lane = lax.iota(jnp.int32, L)
    base_row = wid * RPW

    def row_compute(xv, r):
        def group_body(g, macc_ag):
            macc, agacc = macc_ag
            goff = g * GELEMS

            def ch_body(c, gms):
                g0, g1, g2, g3 = gms
                base = goff + c * (8 * L)
                g0 = jnp.maximum(g0, xv[pl.ds(base + 0 * L, L)])
                g1 = jnp.maximum(g1, xv[pl.ds(base + 1 * L, L)])
                g2 = jnp.maximum(g2, xv[pl.ds(base + 2 * L, L)])
                g3 = jnp.maximum(g3, xv[pl.ds(base + 3 * L, L)])
                g0 = jnp.maximum(g0, xv[pl.ds(base + 4 * L, L)])
                g1 = jnp.maximum(g1, xv[pl.ds(base + 5 * L, L)])
                g2 = jnp.maximum(g2, xv[pl.ds(base + 6 * L, L)])
                g3 = jnp.maximum(g3, xv[pl.ds(base + 7 * L, L)])
                return g0, g1, g2, g3

            g0, g1, g2, g3 = lax.fori_loop(
                0, GCHUNKS // 8, ch_body, (_neg(), _neg(), _neg(), _neg()))
            gm = jnp.maximum(jnp.maximum(g0, g1), jnp.maximum(g2, g3))
            smax[g, :] = gm
            upd = gm > macc
            macc = jnp.where(upd, gm, macc)
            agacc = jnp.where(upd, g, agacc)
            return macc, agacc

        ma, ag = lax.fori_loop(
            0, GROUPS, group_body,
            (_neg(), jnp.zeros((L,), jnp.int32)))

        vacc = jnp.zeros((L,), jnp.float32)
        iacc = jnp.zeros((L,), jnp.int32)
        for k in range(TOPK):
            mk = jnp.max(ma)

            gi = jnp.min(jnp.where(ma == mk, ag, BIG))
            gbase = gi * GELEMS

            def pos_body(c, pa):
                p0, p1, r0, r1, c0, c1 = pa
                base = gbase + c * (8 * L)
                pos = c * (8 * L) + lane
                for j in range(8):
                    vj = xv[pl.ds(base + j * L, L)]
                    ej = vj == mk
                    if j % 2 == 0:
                        p0 = jnp.minimum(p0, jnp.where(ej, pos + j * L, BIG))
                        r0 = jnp.maximum(r0, jnp.where(ej, NEG, vj))
                        c0 = c0 + ej.astype(jnp.int32)
                    else:
                        p1 = jnp.minimum(p1, jnp.where(ej, pos + j * L, BIG))
                        r1 = jnp.maximum(r1, jnp.where(ej, NEG, vj))
                        c1 = c1 + ej.astype(jnp.int32)
                return p0, p1, r0, r1, c0, c1

            bigv = jnp.full((L,), BIG, jnp.int32)
            zi = jnp.zeros((L,), jnp.int32)
            p0, p1, r0, r1, c0, c1 = lax.fori_loop(
                0, GCHUNKS // 8, pos_body,
                (bigv, bigv, _neg(), _neg(), zi, zi))
            e = jnp.min(jnp.minimum(p0, p1))

            vacc = jnp.where(lane == k, mk, vacc)
            iacc = jnp.where(lane == k, gbase + e, iacc)

            if k < TOPK - 1:
                el = e % L
                coff = gbase + e - el
                v = xv[pl.ds(coff, L)]
                xv[pl.ds(coff, L)] = jnp.where(lane == el, NEG, v)

                ct = (c0 + c1) - (lane == el).astype(jnp.int32)
                ru = jnp.maximum(r0, r1)
                smax[gi, :] = jnp.where(ct > 0, mk, ru)

                def remax_body(c, mbs):
                    bs = list(mbs)
                    g = c * 4
                    for j in range(4):
                        b, bg = bs[2 * j], bs[2 * j + 1]
                        gv = smax[g + j, :]
                        u = gv > b
                        bs[2 * j] = jnp.where(u, gv, b)
                        bs[2 * j + 1] = jnp.where(u, g + j, bg)
                    return tuple(bs)

                zi4 = jnp.zeros((L,), jnp.int32)
                st = lax.fori_loop(
                    0, GROUPS // 4, remax_body,
                    (_neg(), zi4, _neg(), zi4, _neg(), zi4, _neg(), zi4))

                def comb(b0, g0, b1, g1):
                    u = (b1 > b0) | ((b1 == b0) & (g1 < g0))
                    return jnp.where(u, b1, b0), jnp.where(u, g1, g0)

                bx, gx = comb(st[0], st[1], st[2], st[3])
                by, gy = comb(st[4], st[5], st[6], st[7])
                ma, ag = comb(bx, gx, by, gy)

        vout[r, :] = vacc
        iout[r, :] = iacc

    pltpu.async_copy(x_hbm.at[base_row], xv0, sem0)
    for r in range(RPW):
        cur, sem_c = (xv0, sem0) if r % 2 == 0 else (xv1, sem1)
        nxt, sem_n = (xv1, sem1) if r % 2 == 0 else (xv0, sem0)
        pltpu.make_async_copy(x_hbm.at[base_row + r], cur, sem_c).wait()
        if r + 1 < RPW:
            pltpu.async_copy(x_hbm.at[base_row + r + 1], nxt, sem_n)
        row_compute(cur, r)

    pltpu.sync_copy(vout, vals_hbm.at[pl.ds(base_row, RPW)])
    pltpu.sync_copy(iout, idx_hbm.at[pl.ds(base_row, RPW)])


def _tc_norm_body(x_ref, m_ref, s_ref):
    xb = x_ref[...]
    m = jnp.max(xb, axis=1, keepdims=True)
    m_ref[...] = m
    s_ref[...] = jnp.sum(jnp.exp(xb - m), axis=1, keepdims=True)


_BR = 16


_tc_norm = pl.pallas_call(
    _tc_norm_body,
    grid=(ROWS // _BR,),
    in_specs=[pl.BlockSpec((_BR, N), lambda i: (i, 0))],
    out_specs=[
        pl.BlockSpec((_BR, 1), lambda i: (i, 0)),
        pl.BlockSpec((_BR, 1), lambda i: (i, 0)),
    ],
    out_shape=[
        jax.ShapeDtypeStruct((ROWS, 1), jnp.float32),
        jax.ShapeDtypeStruct((ROWS, 1), jnp.float32),
    ],
)


@jax.jit
def kernel(x):
    rawv, idx = _sc_topk(x)
    m, s = _tc_norm(x)
    vals = jnp.exp(rawv[:, :TOPK] - m) / s
    return vals, idx[:, :TOPK]
